# baseline (speedup 1.0000x reference)
import functools

import jax
import jax.numpy as jnp
from jax import lax
from jax.experimental import pallas as pl
from jax.experimental.pallas import tpu as pltpu
from jax.experimental.pallas import tpu_sc as plsc


def _project_table_tc(tableT, W2T, bb):
    D, V = tableT.shape
    C = W2T.shape[1]
    NBLK = 16384
    grid = pl.cdiv(V, NBLK)

    def proj(t_ref, w_ref, b_ref, o_ref):
        r = lax.dot_general(
            t_ref[...], w_ref[...], (((0,), (0,)), ((), ())),
            preferred_element_type=jnp.float32)
        o_ref[...] = r + b_ref[...]

    return pl.pallas_call(
        proj,
        grid=(grid,),
        in_specs=[
            pl.BlockSpec((D, NBLK), lambda i: (0, i)),
            pl.BlockSpec((D, C), lambda i: (0, 0)),
            pl.BlockSpec((1, C), lambda i: (0, 0)),
        ],
        out_specs=pl.BlockSpec((NBLK, C), lambda i: (i, 0)),
        out_shape=jax.ShapeDtypeStruct((V, C), jnp.float32),
    )(tableT, W2T, bb)


def _gather_pool_sc(ids2T, p_flat, B, S):
    NW = 32
    b_per_w = B // NW
    bw2 = 2 * b_per_w
    L = 16

    mesh = plsc.VectorSubcoreMesh(core_axis_name="c", subcore_axis_name="s")

    @functools.partial(
        pl.kernel,
        out_type=jax.ShapeDtypeStruct((2 * B,), jnp.float32),
        mesh=mesh,
        scratch_types=[
            pltpu.VMEM((S, bw2), jnp.int32),
            pltpu.VMEM((S * bw2,), jnp.float32),
            pltpu.VMEM((bw2,), jnp.float32),
            pltpu.SemaphoreType.DMA,
        ],
        compiler_params=pltpu.CompilerParams(use_tc_tiling_on_sc=False),
    )
    def gpool(ids_hbm, p_hbm, out_hbm, idx_v, g_v, acc_v, sem):
        wid = lax.axis_index("s") * 2 + lax.axis_index("c")
        base = wid * b_per_w

        pltpu.sync_copy(ids_hbm.at[:, pl.ds(2 * base, bw2)], idx_v)

        def fire(s, carry):
            pltpu.async_copy(
                p_hbm.at[idx_v.at[s, pl.ds(0, b_per_w)]],
                g_v.at[pl.ds(s * bw2, b_per_w)], sem)
            pltpu.async_copy(
                p_hbm.at[idx_v.at[s, pl.ds(b_per_w, b_per_w)]],
                g_v.at[pl.ds(s * bw2 + b_per_w, b_per_w)], sem)
            return carry

        lax.fori_loop(0, S, fire, 0)

        pltpu.make_async_copy(
            p_hbm.at[pl.ds(0, S * bw2)], g_v, sem).wait()

        for jg in range(bw2 // L):
            off = jg * L

            def red(s, acc):
                return acc + g_v[pl.ds(s * bw2 + off, L)]

            acc = lax.fori_loop(0, S, red, jnp.zeros((L,), jnp.float32))
            acc_v[pl.ds(off, L)] = acc

        pltpu.sync_copy(acc_v, out_hbm.at[pl.ds(2 * base, bw2)])

    return gpool(ids2T, p_flat)


@jax.jit
def kernel(input_ids, embed_table, W, b):
    B, S = input_ids.shape
    V = embed_table.shape[0]
    C = W.shape[1]
    idsT = input_ids.astype(jnp.int32).T
    ids2T = jnp.stack([idsT * 2, idsT * 2 + 1], axis=2).reshape(S, 2 * B)
    tableT = embed_table.T
    W2T = W * (1.0 / S)
    bb = (b * (1.0 / S)).reshape(1, C)
    p2 = _project_table_tc(tableT, W2T, bb)
    p_flat = p2.reshape(V * C)
    pooled_flat = _gather_pool_sc(ids2T, p_flat, B, S)
    return pooled_flat.reshape(B, C)


# speedup vs baseline: 4.2566x; 4.2566x over previous
import functools

import jax
import jax.numpy as jnp
from jax import lax
from jax.experimental import pallas as pl
from jax.experimental.pallas import tpu as pltpu
from jax.experimental.pallas import tpu_sc as plsc


def _project_table_tc(tableT, W2, bb):
    C, D = W2.shape
    V = tableT.shape[1]
    NBLK = 65536
    grid = pl.cdiv(V, NBLK)

    def proj(t_ref, w_ref, b_ref, o0_ref, o1_ref):
        r = (
            jnp.dot(w_ref[...], t_ref[...], preferred_element_type=jnp.float32)
            + b_ref[...]
        )
        o0_ref[...] = r[0]
        o1_ref[...] = r[1]

    return pl.pallas_call(
        proj,
        grid=(grid,),
        in_specs=[
            pl.BlockSpec((D, NBLK), lambda i: (0, i)),
            pl.BlockSpec((C, D), lambda i: (0, 0)),
            pl.BlockSpec((C, 1), lambda i: (0, 0)),
        ],
        out_specs=[
            pl.BlockSpec((NBLK,), lambda i: (i,)),
            pl.BlockSpec((NBLK,), lambda i: (i,)),
        ],
        out_shape=[
            jax.ShapeDtypeStruct((V,), jnp.float32),
            jax.ShapeDtypeStruct((V,), jnp.float32),
        ],
    )(tableT, W2, bb)


def _gather_pool_sc(idsT, p0, p1, B, S):
    NW = 32
    b_per_w = B // NW
    L = 16

    mesh = plsc.VectorSubcoreMesh(core_axis_name="c", subcore_axis_name="s")

    @functools.partial(
        pl.kernel,
        out_type=jax.ShapeDtypeStruct((2, B), jnp.float32),
        mesh=mesh,
        scratch_types=[
            pltpu.VMEM((S, b_per_w), jnp.int32),
            pltpu.VMEM((S * b_per_w,), jnp.float32),
            pltpu.VMEM((S * b_per_w,), jnp.float32),
            pltpu.VMEM((2, b_per_w), jnp.float32),
            pltpu.SemaphoreType.DMA,
        ],
        compiler_params=pltpu.CompilerParams(use_tc_tiling_on_sc=False),
    )
    def gpool(ids_hbm, p0_hbm, p1_hbm, out_hbm, idx_v, g0_v, g1_v, out_v, sem):
        wid = lax.axis_index("s") * 2 + lax.axis_index("c")
        base = wid * b_per_w

        pltpu.sync_copy(ids_hbm.at[:, pl.ds(base, b_per_w)], idx_v)

        def fire(s, carry):
            pltpu.async_copy(
                p0_hbm.at[idx_v.at[s]],
                g0_v.at[pl.ds(s * b_per_w, b_per_w)], sem)
            pltpu.async_copy(
                p1_hbm.at[idx_v.at[s]],
                g1_v.at[pl.ds(s * b_per_w, b_per_w)], sem)
            return carry

        lax.fori_loop(0, S, fire, 0)

        pltpu.make_async_copy(
            p0_hbm.at[pl.ds(0, S * b_per_w)], g0_v, sem).wait()
        pltpu.make_async_copy(
            p1_hbm.at[pl.ds(0, S * b_per_w)], g1_v, sem).wait()

        for jg in range(b_per_w // L):
            off = jg * L

            def red(s, accs):
                a0, a1 = accs
                a0 = a0 + g0_v[pl.ds(s * b_per_w + off, L)]
                a1 = a1 + g1_v[pl.ds(s * b_per_w + off, L)]
                return (a0, a1)

            z = jnp.zeros((L,), jnp.float32)
            a0, a1 = lax.fori_loop(0, S, red, (z, z))
            out_v[0, pl.ds(off, L)] = a0
            out_v[1, pl.ds(off, L)] = a1

        pltpu.sync_copy(out_v.at[0], out_hbm.at[0, pl.ds(base, b_per_w)])
        pltpu.sync_copy(out_v.at[1], out_hbm.at[1, pl.ds(base, b_per_w)])

    return gpool(idsT, p0, p1)


@jax.jit
def kernel(input_ids, embed_table, W, b):
    B, S = input_ids.shape
    C = W.shape[1]
    idsT = input_ids.astype(jnp.int32).T
    tableT = embed_table.T
    W2 = (W * (1.0 / S)).T
    bb = (b * (1.0 / S)).reshape(C, 1)
    p0, p1 = _project_table_tc(tableT, W2, bb)
    pooledT = _gather_pool_sc(idsT, p0, p1, B, S)
    return pooledT.T


# speedup vs baseline: 4.2964x; 1.0094x over previous
import functools

import jax
import jax.numpy as jnp
from jax import lax
from jax.experimental import pallas as pl
from jax.experimental.pallas import tpu as pltpu
from jax.experimental.pallas import tpu_sc as plsc


def _project_table_tc(tableT, W2, bb):
    C, D = W2.shape
    V = tableT.shape[1]
    NBLK = 32768
    grid = pl.cdiv(V, NBLK)

    def proj(t_ref, w_ref, b_ref, o0_ref, o1_ref):
        r = (
            jnp.dot(w_ref[...], t_ref[...], preferred_element_type=jnp.float32)
            + b_ref[...]
        )
        o0_ref[...] = r[0]
        o1_ref[...] = r[1]

    return pl.pallas_call(
        proj,
        grid=(grid,),
        in_specs=[
            pl.BlockSpec((D, NBLK), lambda i: (0, i)),
            pl.BlockSpec((C, D), lambda i: (0, 0)),
            pl.BlockSpec((C, 1), lambda i: (0, 0)),
        ],
        out_specs=[
            pl.BlockSpec((NBLK,), lambda i: (i,)),
            pl.BlockSpec((NBLK,), lambda i: (i,)),
        ],
        out_shape=[
            jax.ShapeDtypeStruct((V,), jnp.float32),
            jax.ShapeDtypeStruct((V,), jnp.float32),
        ],
    )(tableT, W2, bb)


def _gather_pool_sc(idsT, p0, p1, B, S):
    NW = 32
    b_per_w = B // NW
    L = 16

    mesh = plsc.VectorSubcoreMesh(core_axis_name="c", subcore_axis_name="s")

    @functools.partial(
        pl.kernel,
        out_type=jax.ShapeDtypeStruct((2, B), jnp.float32),
        mesh=mesh,
        scratch_types=[
            pltpu.VMEM((S, b_per_w), jnp.int32),
            pltpu.VMEM((S * b_per_w,), jnp.float32),
            pltpu.VMEM((S * b_per_w,), jnp.float32),
            pltpu.VMEM((2, b_per_w), jnp.float32),
            pltpu.SemaphoreType.DMA,
        ],
        compiler_params=pltpu.CompilerParams(use_tc_tiling_on_sc=False),
    )
    def gpool(ids_hbm, p0_hbm, p1_hbm, out_hbm, idx_v, g0_v, g1_v, out_v, sem):
        wid = lax.axis_index("s") * 2 + lax.axis_index("c")
        base = wid * b_per_w

        pltpu.sync_copy(ids_hbm.at[:, pl.ds(base, b_per_w)], idx_v)

        def fire(s, carry):
            pltpu.async_copy(
                p0_hbm.at[idx_v.at[s]],
                g0_v.at[pl.ds(s * b_per_w, b_per_w)], sem)
            pltpu.async_copy(
                p1_hbm.at[idx_v.at[s]],
                g1_v.at[pl.ds(s * b_per_w, b_per_w)], sem)
            return carry

        lax.fori_loop(0, S, fire, 0)

        pltpu.make_async_copy(
            p0_hbm.at[pl.ds(0, S * b_per_w)], g0_v, sem).wait()
        pltpu.make_async_copy(
            p1_hbm.at[pl.ds(0, S * b_per_w)], g1_v, sem).wait()

        for jg in range(b_per_w // L):
            off = jg * L

            def red(s, accs):
                a0, a1 = accs
                a0 = a0 + g0_v[pl.ds(s * b_per_w + off, L)]
                a1 = a1 + g1_v[pl.ds(s * b_per_w + off, L)]
                return (a0, a1)

            z = jnp.zeros((L,), jnp.float32)
            a0, a1 = lax.fori_loop(0, S, red, (z, z))
            out_v[0, pl.ds(off, L)] = a0
            out_v[1, pl.ds(off, L)] = a1

        pltpu.sync_copy(out_v.at[0], out_hbm.at[0, pl.ds(base, b_per_w)])
        pltpu.sync_copy(out_v.at[1], out_hbm.at[1, pl.ds(base, b_per_w)])

    return gpool(idsT, p0, p1)


@jax.jit
def kernel(input_ids, embed_table, W, b):
    B, S = input_ids.shape
    C = W.shape[1]
    idsT = input_ids.astype(jnp.int32).T
    tableT = embed_table.T
    W2 = (W * (1.0 / S)).T
    bb = (b * (1.0 / S)).reshape(C, 1)
    p0, p1 = _project_table_tc(tableT, W2, bb)
    pooledT = _gather_pool_sc(idsT, p0, p1, B, S)
    return pooledT.T
